# baseline (speedup 1.0000x reference)
import functools

import jax
import jax.numpy as jnp
from jax import lax
from jax.experimental import pallas as pl
from jax.experimental.pallas import tpu as pltpu

_BB = 16


def _cnn_kernel(x_ref, w1_ref, b1_ref, w2_ref, b2_ref, w3_ref, b3_ref,
                w4_ref, b4_ref, o_ref, acc1, hp_s, p1x, acc2, h2_s, p2):
    f32 = jnp.float32
    bb = o_ref.shape[0]
    r1 = bb * 1024
    m1 = r1 - 128
    r2 = bb * 256
    m2 = r2 - 96

    acc1[pl.ds(0, m1), :] = (
        jnp.dot(x_ref[pl.ds(0, m1), :], w1_ref[0], preferred_element_type=f32)
        + b1_ref[...])
    for i in range(1, 5):
        acc1[pl.ds(0, m1), :] += jnp.dot(
            x_ref[pl.ds(i * 32, m1), :], w1_ref[i], preferred_element_type=f32)
    acc1[pl.ds(m1, 128), :] = jnp.zeros((128, 32), f32)

    def pool1(b, c):
        r0 = pl.multiple_of(b * 1024, 1024)
        a = jnp.maximum(acc1[pl.ds(r0, 1024), :], 0.0)
        hp_s[...] = jnp.max(a.reshape(16, 2, 32, 32), axis=1).reshape(512, 32)
        q0 = pl.multiple_of(b * 256, 256)
        p1x[pl.ds(q0, 256), 0:32] = jnp.maximum(
            hp_s[pl.ds(0, 256, 2), :], hp_s[pl.ds(1, 256, 2), :])
        return c
    lax.fori_loop(0, bb, pool1, 0)

    p1x[pl.ds(r2 - 8, 8), 32:160] = jnp.zeros((8, 128), f32)
    for j in range(1, 5):
        p1x[pl.ds(0, r2 - 8), pl.ds(32 * j, 32)] = p1x[pl.ds(j, r2 - 8), 0:32]

    acc2[pl.ds(0, m2), :] = (
        jnp.dot(p1x[pl.ds(0, m2), :], w2_ref[0], preferred_element_type=f32)
        + b2_ref[...])
    for i in range(1, 5):
        acc2[pl.ds(0, m2), :] += jnp.dot(
            p1x[pl.ds(16 * i, m2), :], w2_ref[i], preferred_element_type=f32)
    acc2[pl.ds(m2, 96), :] = jnp.zeros((96, 64), f32)

    def pool2(b, c):
        r0 = pl.multiple_of(b * 256, 256)
        a2 = jnp.maximum(acc2[pl.ds(r0, 256), :], 0.0)
        h2_s[...] = jnp.max(a2.reshape(8, 2, 16, 64), axis=1).reshape(128, 64)
        q0 = pl.multiple_of(b * 64, 64)
        p2[pl.ds(q0, 64), :] = jnp.maximum(
            h2_s[pl.ds(0, 64, 2), :], h2_s[pl.ds(1, 64, 2), :])
        return c
    lax.fori_loop(0, bb, pool2, 0)

    h = (jnp.dot(p2[pl.ds(0, bb, 64), :], w3_ref[0],
                 preferred_element_type=f32) + b3_ref[...])
    for t in range(1, 25):
        hh, ww = divmod(t, 5)
        h = h + jnp.dot(p2[pl.ds(hh * 8 + ww, bb, 64), :], w3_ref[t],
                        preferred_element_type=f32)
    h = jnp.maximum(h, 0.0)

    o_ref[...] = (jnp.dot(h, w4_ref[...], preferred_element_type=f32)
                  + b4_ref[...])


@jax.jit
def _forward(w1, b1, w2, b2, w3, b3, w4, b4, x_nchw):
    B, C, H, W = x_nchw.shape
    assert (C, H, W) == (3, 32, 32)
    bb = min(_BB, B)
    bp = ((B + bb - 1) // bb) * bb
    x_flat = jnp.transpose(x_nchw, (0, 2, 3, 1)).reshape(B, H * W * C)
    xf = jnp.pad(x_flat.astype(jnp.bfloat16), ((0, bp - B), (0, 12)))
    x15 = jnp.concatenate(
        [xf[:, 3 * j:3 * j + 3072].reshape(bp, 1024, 3) for j in range(5)],
        axis=-1).reshape(bp * 1024, 15)
    w1b = w1.astype(jnp.bfloat16)
    w2k = w2.reshape(5, 5 * 32, 64)

    n_flops = bp * (2 * 896 * 15 * 32 * 5 + 2 * 160 * 160 * 64 * 5
                    + 2 * 25 * 64 * 64 + 2 * 64 * 128)
    n_bytes = 4 * (x15.size + w1.size + w2k.size + w3.size + w4.size + bp * 128)
    out = pl.pallas_call(
        _cnn_kernel,
        out_shape=jax.ShapeDtypeStruct((bp, 128), jnp.float32),
        grid_spec=pltpu.PrefetchScalarGridSpec(
            num_scalar_prefetch=0,
            grid=(bp // bb,),
            in_specs=[
                pl.BlockSpec((bb * 1024, 15), lambda g: (g, 0)),
                pl.BlockSpec((5, 15, 32), lambda g: (0, 0, 0)),
                pl.BlockSpec((1, 32), lambda g: (0, 0)),
                pl.BlockSpec((5, 160, 64), lambda g: (0, 0, 0)),
                pl.BlockSpec((1, 64), lambda g: (0, 0)),
                pl.BlockSpec((25, 64, 64), lambda g: (0, 0, 0)),
                pl.BlockSpec((1, 64), lambda g: (0, 0)),
                pl.BlockSpec((64, 128), lambda g: (0, 0)),
                pl.BlockSpec((1, 128), lambda g: (0, 0)),
            ],
            out_specs=pl.BlockSpec((bb, 128), lambda g: (g, 0)),
            scratch_shapes=[
                pltpu.VMEM((bb * 1024, 32), jnp.float32),
                pltpu.VMEM((512, 32), jnp.float32),
                pltpu.VMEM((bb * 256, 160), jnp.float32),
                pltpu.VMEM((bb * 256, 64), jnp.float32),
                pltpu.VMEM((128, 64), jnp.float32),
                pltpu.VMEM((bb * 64, 64), jnp.float32),
            ],
        ),
        compiler_params=pltpu.CompilerParams(
            dimension_semantics=("parallel",),
            vmem_limit_bytes=64 * 1024 * 1024),
        cost_estimate=pl.CostEstimate(flops=n_flops, transcendentals=0,
                                      bytes_accessed=n_bytes),
    )(x15, w1b, b1, w2k, b2, w3, b3, w4, b4)
    return out[:B, :10]


def kernel(w1, b1, w2, b2, w3, b3, w4, b4, x_nchw):
    return _forward(w1, b1, w2, b2, w3, b3, w4, b4, x_nchw)


# speedup vs baseline: 1.4307x; 1.4307x over previous
import jax
import jax.numpy as jnp
from jax import lax
from jax.experimental import pallas as pl
from jax.experimental.pallas import tpu as pltpu

_BB = 16


def _cnn_kernel(x_ref, w1_ref, b1_ref, w2_ref, b2_ref, w3_ref, b3_ref,
                w4_ref, b4_ref, o_ref, o1_s, hp_s, p1x, acc2, h2_s, p2):
    f32 = jnp.float32
    bb = o_ref.shape[0]
    r2 = bb * 256
    m2 = r2 - 96

    o1_s[...] = jnp.dot(w1_ref[...], x_ref[...], preferred_element_type=f32)

    def pool1(b, c):
        s0 = pl.multiple_of(b * 1024, 1024)
        a = jnp.transpose(o1_s[:, pl.ds(s0, 1024)], (1, 0))
        hp_s[...] = jnp.max(a.reshape(16, 2, 32, 32), axis=1).reshape(512, 32)
        q0 = pl.multiple_of(b * 256, 256)
        praw = jnp.maximum(hp_s[pl.ds(0, 256, 2), :], hp_s[pl.ds(1, 256, 2), :])
        p1x[pl.ds(q0, 256), 0:32] = jnp.maximum(
            praw + b1_ref[...], 0.0).astype(jnp.bfloat16)
        return c
    lax.fori_loop(0, bb, pool1, 0)

    p1x[pl.ds(r2 - 8, 8), 32:160] = jnp.zeros((8, 128), jnp.bfloat16)
    for j in range(1, 5):
        p1x[pl.ds(0, r2 - 8), pl.ds(32 * j, 32)] = p1x[pl.ds(j, r2 - 8), 0:32]

    acc2[pl.ds(0, m2), :] = (
        jnp.dot(p1x[pl.ds(0, m2), :], w2_ref[0], preferred_element_type=f32)
        + b2_ref[...])
    for i in range(1, 5):
        acc2[pl.ds(0, m2), :] += jnp.dot(
            p1x[pl.ds(16 * i, m2), :], w2_ref[i], preferred_element_type=f32)
    acc2[pl.ds(m2, 96), :] = jnp.zeros((96, 64), f32)

    def pool2(b, c):
        r0 = pl.multiple_of(b * 256, 256)
        a2 = jnp.maximum(acc2[pl.ds(r0, 256), :], 0.0)
        h2_s[...] = jnp.max(a2.reshape(8, 2, 16, 64), axis=1).reshape(128, 64)
        q0 = pl.multiple_of(b * 64, 64)
        p2[pl.ds(q0, 64), :] = jnp.maximum(
            h2_s[pl.ds(0, 64, 2), :], h2_s[pl.ds(1, 64, 2), :])
        return c
    lax.fori_loop(0, bb, pool2, 0)

    h = (jnp.dot(p2[pl.ds(0, bb, 64), :], w3_ref[0],
                 preferred_element_type=f32) + b3_ref[...])
    for t in range(1, 25):
        hh, ww = divmod(t, 5)
        h = h + jnp.dot(p2[pl.ds(hh * 8 + ww, bb, 64), :], w3_ref[t],
                        preferred_element_type=f32)
    h = jnp.maximum(h, 0.0)

    o_ref[...] = (jnp.dot(h, w4_ref[...], preferred_element_type=f32)
                  + b4_ref[...])


@jax.jit
def _forward(w1, b1, w2, b2, w3, b3, w4, b4, x_nchw):
    B, C, H, W = x_nchw.shape
    assert (C, H, W) == (3, 32, 32)
    bb = min(_BB, B)
    bp = ((B + bb - 1) // bb) * bb
    x_c = jnp.transpose(x_nchw.reshape(B, C, H * W), (1, 0, 2))
    x_c = x_c.reshape(C, B * H * W).astype(jnp.bfloat16)
    x_c = jnp.pad(x_c, ((0, 0), (0, (bp - B) * 1024 + 160)))
    n_sp = bp * 1024
    x75 = jnp.concatenate(
        [x_c[:, 32 * i + j:32 * i + j + n_sp] for i in range(5)
         for j in range(5)], axis=0)
    w75 = jnp.transpose(w1.reshape(75, 32), (1, 0)).astype(jnp.bfloat16)
    w2k = w2.reshape(5, 5 * 32, 64).astype(jnp.bfloat16)

    n_flops = bp * (2 * 1024 * 75 * 32 + 2 * 160 * 160 * 64 * 5
                    + 2 * 25 * 64 * 64 + 2 * 64 * 128)
    n_bytes = 2 * x75.size + 4 * bp * 128
    out = pl.pallas_call(
        _cnn_kernel,
        out_shape=jax.ShapeDtypeStruct((bp, 128), jnp.float32),
        grid_spec=pltpu.PrefetchScalarGridSpec(
            num_scalar_prefetch=0,
            grid=(bp // bb,),
            in_specs=[
                pl.BlockSpec((75, bb * 1024), lambda g: (0, g)),
                pl.BlockSpec((32, 75), lambda g: (0, 0)),
                pl.BlockSpec((1, 32), lambda g: (0, 0)),
                pl.BlockSpec((5, 160, 64), lambda g: (0, 0, 0)),
                pl.BlockSpec((1, 64), lambda g: (0, 0)),
                pl.BlockSpec((25, 64, 64), lambda g: (0, 0, 0)),
                pl.BlockSpec((1, 64), lambda g: (0, 0)),
                pl.BlockSpec((64, 128), lambda g: (0, 0)),
                pl.BlockSpec((1, 128), lambda g: (0, 0)),
            ],
            out_specs=pl.BlockSpec((bb, 128), lambda g: (g, 0)),
            scratch_shapes=[
                pltpu.VMEM((32, bb * 1024), jnp.float32),
                pltpu.VMEM((512, 32), jnp.float32),
                pltpu.VMEM((bb * 256, 160), jnp.bfloat16),
                pltpu.VMEM((bb * 256, 64), jnp.float32),
                pltpu.VMEM((128, 64), jnp.float32),
                pltpu.VMEM((bb * 64, 64), jnp.float32),
            ],
        ),
        compiler_params=pltpu.CompilerParams(
            dimension_semantics=("parallel",),
            vmem_limit_bytes=64 * 1024 * 1024),
        cost_estimate=pl.CostEstimate(flops=n_flops, transcendentals=0,
                                      bytes_accessed=n_bytes),
    )(x75, w75, b1, w2k, b2, w3, b3, w4, b4)
    return out[:B, :10]


def kernel(w1, b1, w2, b2, w3, b3, w4, b4, x_nchw):
    return _forward(w1, b1, w2, b2, w3, b3, w4, b4, x_nchw)


# speedup vs baseline: 5.8061x; 4.0583x over previous
import jax
import jax.numpy as jnp
from jax import lax
from jax.experimental import pallas as pl
from jax.experimental.pallas import tpu as pltpu

_BB = 16


def _cnn_kernel(x_ref, w1_ref, b1_ref, w2_ref, b2_ref, w3_ref, b3_ref,
                w4_ref, b4_ref, o_ref, x75_s, o1_s, hp_s, p1x, acc2, h2_s, p2):
    f32 = jnp.float32
    bb = o_ref.shape[0]
    r2 = bb * 256
    m2 = r2 - 96

    xb = x_ref[...].astype(jnp.bfloat16)
    for b in range(bb):
        for j in range(5):
            src = xb[b * 3:b * 3 + 3, :]
            rot = jnp.concatenate([src[:, j:], src[:, :j]], axis=1) if j else src
            x75_s[3 * j:3 * j + 3, b * 1024:(b + 1) * 1024] = rot
    v = x75_s[0:15, :]
    for i in range(1, 5):
        s = 32 * i
        x75_s[15 * i:15 * i + 15, :] = jnp.concatenate(
            [v[:, s:], v[:, :s]], axis=1)

    o1_s[...] = jnp.dot(w1_ref[...], x75_s[...], preferred_element_type=f32)

    def pool1(b, c):
        s0 = pl.multiple_of(b * 1024, 1024)
        a = jnp.transpose(o1_s[:, pl.ds(s0, 1024)], (1, 0))
        hp_s[...] = jnp.max(a.reshape(16, 2, 32, 32), axis=1).reshape(512, 32)
        q0 = pl.multiple_of(b * 256, 256)
        praw = jnp.maximum(hp_s[pl.ds(0, 256, 2), :], hp_s[pl.ds(1, 256, 2), :])
        p1x[pl.ds(q0, 256), 0:32] = jnp.maximum(
            praw + b1_ref[...], 0.0).astype(jnp.bfloat16)
        return c
    lax.fori_loop(0, bb, pool1, 0)

    p1x[pl.ds(r2 - 8, 8), 32:160] = jnp.zeros((8, 128), jnp.bfloat16)
    for j in range(1, 5):
        p1x[pl.ds(0, r2 - 8), pl.ds(32 * j, 32)] = p1x[pl.ds(j, r2 - 8), 0:32]

    acc2[pl.ds(0, m2), :] = (
        jnp.dot(p1x[pl.ds(0, m2), :], w2_ref[0], preferred_element_type=f32)
        + b2_ref[...])
    for i in range(1, 5):
        acc2[pl.ds(0, m2), :] += jnp.dot(
            p1x[pl.ds(16 * i, m2), :], w2_ref[i], preferred_element_type=f32)
    acc2[pl.ds(m2, 96), :] = jnp.zeros((96, 64), f32)

    def pool2(b, c):
        r0 = pl.multiple_of(b * 256, 256)
        a2 = jnp.maximum(acc2[pl.ds(r0, 256), :], 0.0)
        h2_s[...] = jnp.max(a2.reshape(8, 2, 16, 64), axis=1).reshape(128, 64)
        q0 = pl.multiple_of(b * 64, 64)
        p2[pl.ds(q0, 64), :] = jnp.maximum(
            h2_s[pl.ds(0, 64, 2), :], h2_s[pl.ds(1, 64, 2), :])
        return c
    lax.fori_loop(0, bb, pool2, 0)

    h = (jnp.dot(p2[pl.ds(0, bb, 64), :], w3_ref[0],
                 preferred_element_type=f32) + b3_ref[...])
    for t in range(1, 25):
        hh, ww = divmod(t, 5)
        h = h + jnp.dot(p2[pl.ds(hh * 8 + ww, bb, 64), :], w3_ref[t],
                        preferred_element_type=f32)
    h = jnp.maximum(h, 0.0)

    o_ref[...] = (jnp.dot(h, w4_ref[...], preferred_element_type=f32)
                  + b4_ref[...])


@jax.jit
def _forward(w1, b1, w2, b2, w3, b3, w4, b4, x_nchw):
    B, C, H, W = x_nchw.shape
    assert (C, H, W) == (3, 32, 32)
    bb = min(_BB, B)
    bp = ((B + bb - 1) // bb) * bb
    x_r = x_nchw.reshape(B * C, H * W)
    if bp != B:
        x_r = jnp.pad(x_r, ((0, (bp - B) * C), (0, 0)))
    w75 = jnp.transpose(w1.reshape(75, 32), (1, 0)).astype(jnp.bfloat16)
    w2k = w2.reshape(5, 5 * 32, 64).astype(jnp.bfloat16)

    n_flops = bp * (2 * 1024 * 75 * 32 + 2 * 160 * 160 * 64 * 5
                    + 2 * 25 * 64 * 64 + 2 * 64 * 128)
    n_bytes = 4 * x_r.size + 4 * bp * 128
    out = pl.pallas_call(
        _cnn_kernel,
        out_shape=jax.ShapeDtypeStruct((bp, 128), jnp.float32),
        grid_spec=pltpu.PrefetchScalarGridSpec(
            num_scalar_prefetch=0,
            grid=(bp // bb,),
            in_specs=[
                pl.BlockSpec((bb * 3, 1024), lambda g: (g, 0)),
                pl.BlockSpec((32, 75), lambda g: (0, 0)),
                pl.BlockSpec((1, 32), lambda g: (0, 0)),
                pl.BlockSpec((5, 160, 64), lambda g: (0, 0, 0)),
                pl.BlockSpec((1, 64), lambda g: (0, 0)),
                pl.BlockSpec((25, 64, 64), lambda g: (0, 0, 0)),
                pl.BlockSpec((1, 64), lambda g: (0, 0)),
                pl.BlockSpec((64, 128), lambda g: (0, 0)),
                pl.BlockSpec((1, 128), lambda g: (0, 0)),
            ],
            out_specs=pl.BlockSpec((bb, 128), lambda g: (g, 0)),
            scratch_shapes=[
                pltpu.VMEM((75, bb * 1024), jnp.bfloat16),
                pltpu.VMEM((32, bb * 1024), jnp.float32),
                pltpu.VMEM((512, 32), jnp.float32),
                pltpu.VMEM((bb * 256, 160), jnp.bfloat16),
                pltpu.VMEM((bb * 256, 64), jnp.float32),
                pltpu.VMEM((128, 64), jnp.float32),
                pltpu.VMEM((bb * 64, 64), jnp.float32),
            ],
        ),
        compiler_params=pltpu.CompilerParams(
            dimension_semantics=("parallel",),
            vmem_limit_bytes=64 * 1024 * 1024),
        cost_estimate=pl.CostEstimate(flops=n_flops, transcendentals=0,
                                      bytes_accessed=n_bytes),
    )(x_r, w75, b1, w2k, b2, w3, b3, w4, b4)
    return out[:B, :10]


def kernel(w1, b1, w2, b2, w3, b3, w4, b4, x_nchw):
    return _forward(w1, b1, w2, b2, w3, b3, w4, b4, x_nchw)


# speedup vs baseline: 6.8032x; 1.1717x over previous
import jax
import jax.numpy as jnp
from jax import lax
from jax.experimental import pallas as pl
from jax.experimental.pallas import tpu as pltpu

_BB = 32


def _cnn_kernel(x_ref, w1_ref, b1_ref, w2_ref, b2_ref, w3_ref, b3_ref,
                w4_ref, b4_ref, o_ref, x75_s, o1_s, hp_s, p1x, h2_s, p2):
    f32 = jnp.float32
    bb = o_ref.shape[0]
    r2 = bb * 256
    m2 = r2 - 96

    xb = x_ref[...].astype(jnp.bfloat16)
    for b in range(bb):
        for j in range(5):
            src = xb[b * 3:b * 3 + 3, :]
            rot = jnp.concatenate([src[:, j:], src[:, :j]], axis=1) if j else src
            x75_s[3 * j:3 * j + 3, b * 1024:(b + 1) * 1024] = rot
    v = x75_s[0:15, :]
    for i in range(1, 5):
        s = 32 * i
        x75_s[15 * i:15 * i + 15, :] = jnp.concatenate(
            [v[:, s:], v[:, :s]], axis=1)

    o1_s[...] = jnp.dot(w1_ref[...], x75_s[...], preferred_element_type=f32)

    def pool1(b, c):
        s0 = pl.multiple_of(b * 1024, 1024)
        a = jnp.transpose(o1_s[:, pl.ds(s0, 1024)], (1, 0))
        hp_s[...] = jnp.max(a.reshape(16, 2, 32, 32), axis=1).reshape(512, 32)
        q0 = pl.multiple_of(b * 256, 256)
        praw = jnp.maximum(hp_s[pl.ds(0, 256, 2), :], hp_s[pl.ds(1, 256, 2), :])
        p1x[pl.ds(q0, 256), 0:32] = jnp.maximum(
            praw + b1_ref[...], 0.0).astype(jnp.bfloat16)
        return c
    lax.fori_loop(0, bb, pool1, 0)

    p1x[pl.ds(r2 - 8, 8), 32:160] = jnp.zeros((8, 128), jnp.bfloat16)
    for j in range(1, 5):
        p1x[pl.ds(0, r2 - 8), pl.ds(32 * j, 32)] = p1x[pl.ds(j, r2 - 8), 0:32]

    p1x[pl.ds(r2, 64), :] = jnp.zeros((64, 160), jnp.bfloat16)
    for c in range(bb // 2):
        r0 = c * 512
        z = (jnp.dot(p1x[r0:r0 + 512, :], w2_ref[0],
                     preferred_element_type=f32) + b2_ref[...])
        for i in range(1, 5):
            z = z + jnp.dot(p1x[r0 + 16 * i:r0 + 16 * i + 512, :], w2_ref[i],
                            preferred_element_type=f32)
        a2 = jnp.maximum(z, 0.0)
        h2_s[...] = jnp.max(a2.reshape(16, 2, 16, 64), axis=1).reshape(256, 64)
        p2[128 * c:128 * (c + 1), :] = jnp.maximum(
            h2_s[pl.ds(0, 128, 2), :], h2_s[pl.ds(1, 128, 2), :])

    h = (jnp.dot(p2[pl.ds(0, bb, 64), :], w3_ref[0],
                 preferred_element_type=f32) + b3_ref[...])
    for t in range(1, 25):
        hh, ww = divmod(t, 5)
        h = h + jnp.dot(p2[pl.ds(hh * 8 + ww, bb, 64), :], w3_ref[t],
                        preferred_element_type=f32)
    h = jnp.maximum(h, 0.0)

    o_ref[...] = (jnp.dot(h, w4_ref[...], preferred_element_type=f32)
                  + b4_ref[...])


@jax.jit
def _forward(w1, b1, w2, b2, w3, b3, w4, b4, x_nchw):
    B, C, H, W = x_nchw.shape
    assert (C, H, W) == (3, 32, 32)
    bb = min(_BB, B)
    bp = ((B + bb - 1) // bb) * bb
    x_r = x_nchw.reshape(B * C, H * W)
    if bp != B:
        x_r = jnp.pad(x_r, ((0, (bp - B) * C), (0, 0)))
    w75 = jnp.transpose(w1.reshape(75, 32), (1, 0)).astype(jnp.bfloat16)
    w2k = w2.reshape(5, 5 * 32, 64).astype(jnp.bfloat16)

    n_flops = bp * (2 * 1024 * 75 * 32 + 2 * 160 * 160 * 64 * 5
                    + 2 * 25 * 64 * 64 + 2 * 64 * 128)
    n_bytes = 4 * x_r.size + 4 * bp * 128
    out = pl.pallas_call(
        _cnn_kernel,
        out_shape=jax.ShapeDtypeStruct((bp, 128), jnp.float32),
        grid_spec=pltpu.PrefetchScalarGridSpec(
            num_scalar_prefetch=0,
            grid=(bp // bb,),
            in_specs=[
                pl.BlockSpec((bb * 3, 1024), lambda g: (g, 0)),
                pl.BlockSpec((32, 75), lambda g: (0, 0)),
                pl.BlockSpec((1, 32), lambda g: (0, 0)),
                pl.BlockSpec((5, 160, 64), lambda g: (0, 0, 0)),
                pl.BlockSpec((1, 64), lambda g: (0, 0)),
                pl.BlockSpec((25, 64, 64), lambda g: (0, 0, 0)),
                pl.BlockSpec((1, 64), lambda g: (0, 0)),
                pl.BlockSpec((64, 128), lambda g: (0, 0)),
                pl.BlockSpec((1, 128), lambda g: (0, 0)),
            ],
            out_specs=pl.BlockSpec((bb, 128), lambda g: (g, 0)),
            scratch_shapes=[
                pltpu.VMEM((75, bb * 1024), jnp.bfloat16),
                pltpu.VMEM((32, bb * 1024), jnp.float32),
                pltpu.VMEM((512, 32), jnp.float32),
                pltpu.VMEM((bb * 256 + 64, 160), jnp.bfloat16),
                pltpu.VMEM((256, 64), jnp.float32),
                pltpu.VMEM((bb * 64, 64), jnp.float32),
            ],
        ),
        compiler_params=pltpu.CompilerParams(
            dimension_semantics=("parallel",),
            vmem_limit_bytes=64 * 1024 * 1024),
        cost_estimate=pl.CostEstimate(flops=n_flops, transcendentals=0,
                                      bytes_accessed=n_bytes),
    )(x_r, w75, b1, w2k, b2, w3, b3, w4, b4)
    return out[:B, :10]


def kernel(w1, b1, w2, b2, w3, b3, w4, b4, x_nchw):
    return _forward(w1, b1, w2, b2, w3, b3, w4, b4, x_nchw)


# speedup vs baseline: 6.9645x; 1.0237x over previous
import jax
import jax.numpy as jnp
from jax import lax
from jax.experimental import pallas as pl
from jax.experimental.pallas import tpu as pltpu

_BB = 64


def _cnn_kernel(x_ref, w1_ref, b1_ref, w2_ref, b2_ref, w3_ref, b3_ref,
                w4_ref, b4_ref, o_ref, x75_s, o1_s, hp_s, p1x, h2_s, p2):
    f32 = jnp.float32
    bb = o_ref.shape[0]
    r2 = bb * 256
    m2 = r2 - 96

    xb = x_ref[...].astype(jnp.bfloat16)
    for b in range(bb):
        for j in range(5):
            src = xb[b * 3:b * 3 + 3, :]
            rot = jnp.concatenate([src[:, j:], src[:, :j]], axis=1) if j else src
            x75_s[3 * j:3 * j + 3, b * 1024:(b + 1) * 1024] = rot
    v = x75_s[0:15, :]
    for i in range(1, 5):
        s = 32 * i
        x75_s[15 * i:15 * i + 15, :] = jnp.concatenate(
            [v[:, s:], v[:, :s]], axis=1)

    o1_s[...] = jnp.dot(w1_ref[...], x75_s[...], preferred_element_type=f32)

    def pool1(b, c):
        s0 = pl.multiple_of(b * 1024, 1024)
        a = jnp.transpose(o1_s[:, pl.ds(s0, 1024)], (1, 0))
        hp_s[...] = jnp.max(a.reshape(16, 2, 32, 32), axis=1).reshape(512, 32)
        q0 = pl.multiple_of(b * 256, 256)
        praw = jnp.maximum(hp_s[pl.ds(0, 256, 2), :], hp_s[pl.ds(1, 256, 2), :])
        p1x[pl.ds(q0, 256), 0:32] = jnp.maximum(
            praw + b1_ref[...], 0.0).astype(jnp.bfloat16)
        return c
    lax.fori_loop(0, bb, pool1, 0)

    p1x[pl.ds(r2 - 8, 8), 32:160] = jnp.zeros((8, 128), jnp.bfloat16)
    for j in range(1, 5):
        p1x[pl.ds(0, r2 - 8), pl.ds(32 * j, 32)] = p1x[pl.ds(j, r2 - 8), 0:32]

    p1x[pl.ds(r2, 64), :] = jnp.zeros((64, 160), jnp.bfloat16)
    for c in range(bb // 2):
        r0 = c * 512
        z = (jnp.dot(p1x[r0:r0 + 512, :], w2_ref[0],
                     preferred_element_type=f32) + b2_ref[...])
        for i in range(1, 5):
            z = z + jnp.dot(p1x[r0 + 16 * i:r0 + 16 * i + 512, :], w2_ref[i],
                            preferred_element_type=f32)
        a2 = jnp.maximum(z, 0.0)
        h2_s[...] = jnp.max(a2.reshape(16, 2, 16, 64), axis=1).reshape(256, 64)
        p2[128 * c:128 * (c + 1), :] = jnp.maximum(
            h2_s[pl.ds(0, 128, 2), :], h2_s[pl.ds(1, 128, 2), :])

    h = (jnp.dot(p2[pl.ds(0, bb, 64), :], w3_ref[0],
                 preferred_element_type=f32) + b3_ref[...])
    for t in range(1, 25):
        hh, ww = divmod(t, 5)
        h = h + jnp.dot(p2[pl.ds(hh * 8 + ww, bb, 64), :], w3_ref[t],
                        preferred_element_type=f32)
    h = jnp.maximum(h, 0.0)

    o_ref[...] = (jnp.dot(h, w4_ref[...], preferred_element_type=f32)
                  + b4_ref[...])


@jax.jit
def _forward(w1, b1, w2, b2, w3, b3, w4, b4, x_nchw):
    B, C, H, W = x_nchw.shape
    assert (C, H, W) == (3, 32, 32)
    bb = min(_BB, B)
    bp = ((B + bb - 1) // bb) * bb
    x_r = x_nchw.reshape(B * C, H * W)
    if bp != B:
        x_r = jnp.pad(x_r, ((0, (bp - B) * C), (0, 0)))
    w75 = jnp.transpose(w1.reshape(75, 32), (1, 0)).astype(jnp.bfloat16)
    w2k = w2.reshape(5, 5 * 32, 64).astype(jnp.bfloat16)

    n_flops = bp * (2 * 1024 * 75 * 32 + 2 * 160 * 160 * 64 * 5
                    + 2 * 25 * 64 * 64 + 2 * 64 * 128)
    n_bytes = 4 * x_r.size + 4 * bp * 128
    out = pl.pallas_call(
        _cnn_kernel,
        out_shape=jax.ShapeDtypeStruct((bp, 128), jnp.float32),
        grid_spec=pltpu.PrefetchScalarGridSpec(
            num_scalar_prefetch=0,
            grid=(bp // bb,),
            in_specs=[
                pl.BlockSpec((bb * 3, 1024), lambda g: (g, 0)),
                pl.BlockSpec((32, 75), lambda g: (0, 0)),
                pl.BlockSpec((1, 32), lambda g: (0, 0)),
                pl.BlockSpec((5, 160, 64), lambda g: (0, 0, 0)),
                pl.BlockSpec((1, 64), lambda g: (0, 0)),
                pl.BlockSpec((25, 64, 64), lambda g: (0, 0, 0)),
                pl.BlockSpec((1, 64), lambda g: (0, 0)),
                pl.BlockSpec((64, 128), lambda g: (0, 0)),
                pl.BlockSpec((1, 128), lambda g: (0, 0)),
            ],
            out_specs=pl.BlockSpec((bb, 128), lambda g: (g, 0)),
            scratch_shapes=[
                pltpu.VMEM((75, bb * 1024), jnp.bfloat16),
                pltpu.VMEM((32, bb * 1024), jnp.float32),
                pltpu.VMEM((512, 32), jnp.float32),
                pltpu.VMEM((bb * 256 + 64, 160), jnp.bfloat16),
                pltpu.VMEM((256, 64), jnp.float32),
                pltpu.VMEM((bb * 64, 64), jnp.float32),
            ],
        ),
        compiler_params=pltpu.CompilerParams(
            dimension_semantics=("parallel",),
            vmem_limit_bytes=64 * 1024 * 1024),
        cost_estimate=pl.CostEstimate(flops=n_flops, transcendentals=0,
                                      bytes_accessed=n_bytes),
    )(x_r, w75, b1, w2k, b2, w3, b3, w4, b4)
    return out[:B, :10]


def kernel(w1, b1, w2, b2, w3, b3, w4, b4, x_nchw):
    return _forward(w1, b1, w2, b2, w3, b3, w4, b4, x_nchw)
